# Initial kernel scaffold; baseline (speedup 1.0000x reference)
import jax
import jax.numpy as jnp
from jax.experimental import pallas as pl


def kernel(x, W1, b1, W2, b2, edge_index):
    raise NotImplementedError("write your pallas kernel here")


# speedup vs baseline: 8.5577x; 8.5577x over previous
import functools

import jax
import jax.numpy as jnp
from jax import lax
from jax.experimental import pallas as pl
from jax.experimental.pallas import tpu as pltpu
from jax.experimental.pallas import tpu_sc as plsc

N = 10000
F = 256
HALF = 128
E = 160000
NCHUNK = 32
CHUNK = 5000
CHUNK_PAD = 5120
NB = 40
BATCH = 128
ACC_ROWS = 10016
ROWS_PER_TILE = 632
LAST_ROWS = N - 15 * ROWS_PER_TILE
DEG_ROWS = 80
ROW_BLK = 1000

_mesh = plsc.VectorSubcoreMesh(core_axis_name="c", subcore_axis_name="s")


HIST = DEG_ROWS * BATCH


@functools.partial(
    pl.kernel,
    mesh=_mesh,
    out_type=jax.ShapeDtypeStruct((NCHUNK, HIST), jnp.float32),
    scratch_types=[
        pltpu.VMEM((NB, BATCH), jnp.int32),
        pltpu.VMEM((HIST,), jnp.float32),
    ],
    compiler_params=pltpu.CompilerParams(needs_layout_passes=False),
)
def _sc_deg(dst_hbm, d_hbm, dst_v, hist_v):
    c = lax.axis_index("c")
    s = lax.axis_index("s")
    chunk = c * 16 + s
    pltpu.sync_copy(dst_hbm.at[chunk], dst_v)

    zeros16 = jnp.zeros((16,), jnp.float32)
    ones16 = jnp.ones((16,), jnp.float32)

    def zero_body(i, carry):
        hist_v[pl.ds(i * 16, 16)] = zeros16
        return carry
    lax.fori_loop(0, HIST // 16, zero_body, 0)

    def acc_body(i, carry):
        idx = dst_v[i // 8, pl.ds((i % 8) * 16, 16)]
        plsc.addupdate_scatter(hist_v, [idx], ones16)
        return carry
    lax.fori_loop(0, NB * 8, acc_body, 0)

    pltpu.sync_copy(hist_v, d_hbm.at[chunk])


@functools.partial(
    pl.kernel,
    mesh=_mesh,
    out_type=(
        jax.ShapeDtypeStruct((N, HALF), jnp.float32),
        jax.ShapeDtypeStruct((N, HALF), jnp.float32),
    ),
    scratch_types=[
        pltpu.VMEM((NB, BATCH), jnp.int32),
        pltpu.VMEM((NB, BATCH), jnp.int32),
        pltpu.VMEM((BATCH, HALF), jnp.float32),
        pltpu.VMEM_SHARED((ACC_ROWS, HALF), jnp.float32),
        pltpu.SemaphoreType.DMA,
    ],
    compiler_params=pltpu.CompilerParams(needs_layout_passes=False),
)
def _sc_edge(h0_hbm, h1_hbm, src_hbm, dst_hbm, o0_hbm, o1_hbm,
             src_v, dst_v, buf_v, acc_sh, sem):
    c = lax.axis_index("c")
    s = lax.axis_index("s")

    def run(h_hbm, o_hbm):
        @pl.when(s < 15)
        def _():
            pltpu.sync_copy(h_hbm.at[pl.ds(s * ROWS_PER_TILE, ROWS_PER_TILE)],
                            acc_sh.at[pl.ds(s * ROWS_PER_TILE, ROWS_PER_TILE)])

        @pl.when(s == 15)
        def _():
            pltpu.sync_copy(h_hbm.at[pl.ds(15 * ROWS_PER_TILE, LAST_ROWS)],
                            acc_sh.at[pl.ds(15 * ROWS_PER_TILE, LAST_ROWS)])
        plsc.subcore_barrier()
        for half in range(2):
            chunk = s + 16 * half
            pltpu.sync_copy(src_hbm.at[chunk], src_v)
            pltpu.sync_copy(dst_hbm.at[chunk], dst_v)

            def body(j, carry):
                pltpu.async_copy(h_hbm.at[src_v.at[j]], buf_v, sem).wait()
                pltpu.sync_copy(buf_v, acc_sh.at[dst_v.at[j]], add=True)
                return carry
            lax.fori_loop(0, NB, body, 0)
        plsc.subcore_barrier()

        @pl.when(s < 15)
        def _():
            pltpu.sync_copy(acc_sh.at[pl.ds(s * ROWS_PER_TILE, ROWS_PER_TILE)],
                            o_hbm.at[pl.ds(s * ROWS_PER_TILE, ROWS_PER_TILE)])

        @pl.when(s == 15)
        def _():
            pltpu.sync_copy(acc_sh.at[pl.ds(15 * ROWS_PER_TILE, LAST_ROWS)],
                            o_hbm.at[pl.ds(15 * ROWS_PER_TILE, LAST_ROWS)])

    @pl.when(c == 0)
    def _():
        run(h0_hbm, o0_hbm)

    @pl.when(c == 1)
    def _():
        run(h1_hbm, o1_hbm)


def _tc1_body(x_ref, w_ref, d_ref, h0_ref, h1_ref, dv_ref):
    deg = jnp.sum(d_ref[:], axis=1, keepdims=True) + 1.0
    dv = lax.rsqrt(deg)
    h = jnp.dot(x_ref[:], w_ref[:], preferred_element_type=jnp.float32)
    h = h * dv
    h0_ref[:] = h[:, :HALF]
    h1_ref[:] = h[:, HALF:]
    dv_ref[:] = dv


def _tc2_body(a0_ref, a1_ref, dv_ref, b1_ref, w2_ref, g0_ref, g1_ref):
    dv = dv_ref[:]
    xb = jnp.concatenate([a0_ref[:], a1_ref[:]], axis=1) * dv + b1_ref[:][None, :]
    xb = jnp.maximum(xb, 0.0)
    g = jnp.dot(xb, w2_ref[:], preferred_element_type=jnp.float32) * dv
    g0_ref[:] = g[:, :HALF]
    g1_ref[:] = g[:, HALF:]


def _tc3_body(a0_ref, a1_ref, dv_ref, b2_ref, o_ref):
    o_ref[:] = (jnp.concatenate([a0_ref[:], a1_ref[:]], axis=1) * dv_ref[:]
                + b2_ref[:][None, :])


_GRID = N // ROW_BLK

_row_spec = pl.BlockSpec((ROW_BLK, F), lambda i: (i, 0))
_half_spec = pl.BlockSpec((ROW_BLK, HALF), lambda i: (i, 0))
_vec_spec = pl.BlockSpec((ROW_BLK, 1), lambda i: (i, 0))
_w_spec = pl.BlockSpec((F, F), lambda i: (0, 0))
_b_spec = pl.BlockSpec((F,), lambda i: (0,))

_deg_spec = pl.BlockSpec((ROW_BLK, NCHUNK), lambda i: (i, 0))

_tc1 = pl.pallas_call(
    _tc1_body,
    grid=(_GRID,),
    in_specs=[_row_spec, _w_spec, _deg_spec],
    out_specs=(_half_spec, _half_spec, _vec_spec),
    out_shape=(
        jax.ShapeDtypeStruct((N, HALF), jnp.float32),
        jax.ShapeDtypeStruct((N, HALF), jnp.float32),
        jax.ShapeDtypeStruct((N, 1), jnp.float32),
    ),
)

_tc2 = pl.pallas_call(
    _tc2_body,
    grid=(_GRID,),
    in_specs=[_half_spec, _half_spec, _vec_spec, _b_spec, _w_spec],
    out_specs=(_half_spec, _half_spec),
    out_shape=(
        jax.ShapeDtypeStruct((N, HALF), jnp.float32),
        jax.ShapeDtypeStruct((N, HALF), jnp.float32),
    ),
)

_tc3 = pl.pallas_call(
    _tc3_body,
    grid=(_GRID,),
    in_specs=[_half_spec, _half_spec, _vec_spec, _b_spec],
    out_specs=_row_spec,
    out_shape=jax.ShapeDtypeStruct((N, F), jnp.float32),
)


def kernel(x, W1, b1, W2, b2, edge_index):
    src = edge_index[0].astype(jnp.int32)
    dst = edge_index[1].astype(jnp.int32)
    srcp = jnp.pad(src.reshape(NCHUNK, CHUNK),
                   ((0, 0), (0, CHUNK_PAD - CHUNK))).reshape(NCHUNK, NB, BATCH)
    dstp = jnp.pad(dst.reshape(NCHUNK, CHUNK),
                   ((0, 0), (0, CHUNK_PAD - CHUNK)),
                   constant_values=N).reshape(NCHUNK, NB, BATCH)

    dparts = _sc_deg(dstp)
    dparts = dparts.T[:N]
    h0, h1, dv = _tc1(x, W1, dparts)
    a0, a1 = _sc_edge(h0, h1, srcp, dstp)
    g0, g1 = _tc2(a0, a1, dv, b1, W2)
    o0, o1 = _sc_edge(g0, g1, srcp, dstp)
    return _tc3(o0, o1, dv, b2)


# speedup vs baseline: 9.6472x; 1.1273x over previous
import functools

import jax
import jax.numpy as jnp
from jax import lax
from jax.experimental import pallas as pl
from jax.experimental.pallas import tpu as pltpu
from jax.experimental.pallas import tpu_sc as plsc

N = 10000
F = 256
HALF = 128
E = 160000
NCHUNK = 32
CHUNK = 5000
CHUNK_PAD = 5120
NB = 40
BATCH = 128
ACC_ROWS = 10016
ROWS_PER_TILE = 632
LAST_ROWS = N - 15 * ROWS_PER_TILE
DEG_ROWS = 80
ROW_BLK = 1000

_mesh = plsc.VectorSubcoreMesh(core_axis_name="c", subcore_axis_name="s")


HIST = DEG_ROWS * BATCH


@functools.partial(
    pl.kernel,
    mesh=_mesh,
    out_type=jax.ShapeDtypeStruct((NCHUNK, HIST), jnp.float32),
    scratch_types=[
        pltpu.VMEM((NB, BATCH), jnp.int32),
        pltpu.VMEM((HIST,), jnp.float32),
    ],
    compiler_params=pltpu.CompilerParams(needs_layout_passes=False),
)
def _sc_deg(dst_hbm, d_hbm, dst_v, hist_v):
    c = lax.axis_index("c")
    s = lax.axis_index("s")
    chunk = c * 16 + s
    pltpu.sync_copy(dst_hbm.at[chunk], dst_v)

    zeros16 = jnp.zeros((16,), jnp.float32)
    ones16 = jnp.ones((16,), jnp.float32)

    def zero_body(i, carry):
        hist_v[pl.ds(i * 16, 16)] = zeros16
        return carry
    lax.fori_loop(0, HIST // 16, zero_body, 0)

    def acc_body(i, carry):
        idx = dst_v[i // 8, pl.ds((i % 8) * 16, 16)]
        plsc.addupdate_scatter(hist_v, [idx], ones16)
        return carry
    lax.fori_loop(0, NB * 8, acc_body, 0)

    pltpu.sync_copy(hist_v, d_hbm.at[chunk])


@functools.partial(
    pl.kernel,
    mesh=_mesh,
    out_type=(
        jax.ShapeDtypeStruct((N, HALF), jnp.float32),
        jax.ShapeDtypeStruct((N, HALF), jnp.float32),
    ),
    scratch_types=[
        pltpu.VMEM((NB, BATCH), jnp.int32),
        pltpu.VMEM((NB, BATCH), jnp.int32),
        pltpu.VMEM((2, BATCH, HALF), jnp.float32),
        pltpu.VMEM_SHARED((ACC_ROWS, HALF), jnp.float32),
        pltpu.SemaphoreType.DMA((2,)),
        pltpu.SemaphoreType.DMA((2,)),
    ],
    compiler_params=pltpu.CompilerParams(needs_layout_passes=False),
)
def _sc_edge(h0_hbm, h1_hbm, src_hbm, dst_hbm, o0_hbm, o1_hbm,
             src_v, dst_v, buf_v, acc_sh, gsem, ssem):
    c = lax.axis_index("c")
    s = lax.axis_index("s")

    def run(h_hbm, o_hbm):
        @pl.when(s < 15)
        def _():
            pltpu.sync_copy(h_hbm.at[pl.ds(s * ROWS_PER_TILE, ROWS_PER_TILE)],
                            acc_sh.at[pl.ds(s * ROWS_PER_TILE, ROWS_PER_TILE)])

        @pl.when(s == 15)
        def _():
            pltpu.sync_copy(h_hbm.at[pl.ds(15 * ROWS_PER_TILE, LAST_ROWS)],
                            acc_sh.at[pl.ds(15 * ROWS_PER_TILE, LAST_ROWS)])
        plsc.subcore_barrier()

        def g_desc(j, slot):
            return pltpu.make_async_copy(
                h_hbm.at[src_v.at[j]], buf_v.at[slot], gsem.at[slot])

        def s_desc(j, slot):
            return pltpu.make_async_copy(
                buf_v.at[slot], acc_sh.at[dst_v.at[j]], ssem.at[slot])

        def do_chunk(chunk):
            pltpu.sync_copy(src_hbm.at[chunk], src_v)
            pltpu.sync_copy(dst_hbm.at[chunk], dst_v)
            g_desc(0, 0).start()

            def body(j, carry):
                slot = lax.bitwise_and(j, 1)
                other = 1 - slot
                g_desc(j, slot).wait()
                s_desc(j, slot).start(add=True)

                @pl.when(j >= 1)
                def _():
                    s_desc(j - 1, other).wait()

                @pl.when(j + 1 < NB)
                def _():
                    g_desc(j + 1, other).start()
                return carry
            lax.fori_loop(0, NB, body, 0)
            s_desc(NB - 1, (NB - 1) & 1).wait()

        do_chunk(s)
        do_chunk(s + 16)
        plsc.subcore_barrier()

        @pl.when(s < 15)
        def _():
            pltpu.sync_copy(acc_sh.at[pl.ds(s * ROWS_PER_TILE, ROWS_PER_TILE)],
                            o_hbm.at[pl.ds(s * ROWS_PER_TILE, ROWS_PER_TILE)])

        @pl.when(s == 15)
        def _():
            pltpu.sync_copy(acc_sh.at[pl.ds(15 * ROWS_PER_TILE, LAST_ROWS)],
                            o_hbm.at[pl.ds(15 * ROWS_PER_TILE, LAST_ROWS)])

    @pl.when(c == 0)
    def _():
        run(h0_hbm, o0_hbm)

    @pl.when(c == 1)
    def _():
        run(h1_hbm, o1_hbm)


def _tc1_body(x_ref, w_ref, d_ref, h0_ref, h1_ref, dv_ref):
    deg = jnp.sum(d_ref[:], axis=1, keepdims=True) + 1.0
    dv = lax.rsqrt(deg)
    h = jnp.dot(x_ref[:], w_ref[:], preferred_element_type=jnp.float32)
    h = h * dv
    h0_ref[:] = h[:, :HALF]
    h1_ref[:] = h[:, HALF:]
    dv_ref[:] = dv


def _tc2_body(a0_ref, a1_ref, dv_ref, b1_ref, w2_ref, g0_ref, g1_ref):
    dv = dv_ref[:]
    xb = jnp.concatenate([a0_ref[:], a1_ref[:]], axis=1) * dv + b1_ref[:][None, :]
    xb = jnp.maximum(xb, 0.0)
    g = jnp.dot(xb, w2_ref[:], preferred_element_type=jnp.float32) * dv
    g0_ref[:] = g[:, :HALF]
    g1_ref[:] = g[:, HALF:]


def _tc3_body(a0_ref, a1_ref, dv_ref, b2_ref, o_ref):
    o_ref[:] = (jnp.concatenate([a0_ref[:], a1_ref[:]], axis=1) * dv_ref[:]
                + b2_ref[:][None, :])


_GRID = N // ROW_BLK

_row_spec = pl.BlockSpec((ROW_BLK, F), lambda i: (i, 0))
_half_spec = pl.BlockSpec((ROW_BLK, HALF), lambda i: (i, 0))
_vec_spec = pl.BlockSpec((ROW_BLK, 1), lambda i: (i, 0))
_w_spec = pl.BlockSpec((F, F), lambda i: (0, 0))
_b_spec = pl.BlockSpec((F,), lambda i: (0,))

_deg_spec = pl.BlockSpec((ROW_BLK, NCHUNK), lambda i: (i, 0))

_tc1 = pl.pallas_call(
    _tc1_body,
    grid=(_GRID,),
    in_specs=[_row_spec, _w_spec, _deg_spec],
    out_specs=(_half_spec, _half_spec, _vec_spec),
    out_shape=(
        jax.ShapeDtypeStruct((N, HALF), jnp.float32),
        jax.ShapeDtypeStruct((N, HALF), jnp.float32),
        jax.ShapeDtypeStruct((N, 1), jnp.float32),
    ),
)

_tc2 = pl.pallas_call(
    _tc2_body,
    grid=(_GRID,),
    in_specs=[_half_spec, _half_spec, _vec_spec, _b_spec, _w_spec],
    out_specs=(_half_spec, _half_spec),
    out_shape=(
        jax.ShapeDtypeStruct((N, HALF), jnp.float32),
        jax.ShapeDtypeStruct((N, HALF), jnp.float32),
    ),
)

_tc3 = pl.pallas_call(
    _tc3_body,
    grid=(_GRID,),
    in_specs=[_half_spec, _half_spec, _vec_spec, _b_spec],
    out_specs=_row_spec,
    out_shape=jax.ShapeDtypeStruct((N, F), jnp.float32),
)


def kernel(x, W1, b1, W2, b2, edge_index):
    src = edge_index[0].astype(jnp.int32)
    dst = edge_index[1].astype(jnp.int32)
    srcp = jnp.pad(src.reshape(NCHUNK, CHUNK),
                   ((0, 0), (0, CHUNK_PAD - CHUNK))).reshape(NCHUNK, NB, BATCH)
    dstp = jnp.pad(dst.reshape(NCHUNK, CHUNK),
                   ((0, 0), (0, CHUNK_PAD - CHUNK)),
                   constant_values=N).reshape(NCHUNK, NB, BATCH)

    dparts = _sc_deg(dstp)
    dparts = dparts.T[:N]
    h0, h1, dv = _tc1(x, W1, dparts)
    a0, a1 = _sc_edge(h0, h1, srcp, dstp)
    g0, g1 = _tc2(a0, a1, dv, b1, W2)
    o0, o1 = _sc_edge(g0, g1, srcp, dstp)
    return _tc3(o0, o1, dv, b2)


# speedup vs baseline: 9.6875x; 1.0042x over previous
import functools

import jax
import jax.numpy as jnp
from jax import lax
from jax.experimental import pallas as pl
from jax.experimental.pallas import tpu as pltpu
from jax.experimental.pallas import tpu_sc as plsc

N = 10000
F = 256
HALF = 128
E = 160000
NCHUNK = 32
CHUNK = 5000
CHUNK_PAD = 5120
NB = 40
BATCH = 128
ACC_ROWS = 10016
ROWS_PER_TILE = 632
LAST_ROWS = N - 15 * ROWS_PER_TILE
GBATCH = 64
GNB = CHUNK_PAD // GBATCH
NBUF = 2
DEG_ROWS = 80
ROW_BLK = 1000

_mesh = plsc.VectorSubcoreMesh(core_axis_name="c", subcore_axis_name="s")


HIST = DEG_ROWS * BATCH


@functools.partial(
    pl.kernel,
    mesh=_mesh,
    out_type=jax.ShapeDtypeStruct((NCHUNK, HIST), jnp.float32),
    scratch_types=[
        pltpu.VMEM((NB, BATCH), jnp.int32),
        pltpu.VMEM((HIST,), jnp.float32),
    ],
    compiler_params=pltpu.CompilerParams(needs_layout_passes=False),
)
def _sc_deg(dst_hbm, d_hbm, dst_v, hist_v):
    c = lax.axis_index("c")
    s = lax.axis_index("s")
    chunk = c * 16 + s
    pltpu.sync_copy(dst_hbm.at[chunk], dst_v)

    zeros16 = jnp.zeros((16,), jnp.float32)
    ones16 = jnp.ones((16,), jnp.float32)

    def zero_body(i, carry):
        hist_v[pl.ds(i * 16, 16)] = zeros16
        return carry
    lax.fori_loop(0, HIST // 16, zero_body, 0)

    def acc_body(i, carry):
        idx = dst_v[i // 8, pl.ds((i % 8) * 16, 16)]
        plsc.addupdate_scatter(hist_v, [idx], ones16)
        return carry
    lax.fori_loop(0, NB * 8, acc_body, 0)

    pltpu.sync_copy(hist_v, d_hbm.at[chunk])


@functools.partial(
    pl.kernel,
    mesh=_mesh,
    out_type=(
        jax.ShapeDtypeStruct((N, HALF), jnp.float32),
        jax.ShapeDtypeStruct((N, HALF), jnp.float32),
    ),
    scratch_types=[
        pltpu.VMEM((GNB, GBATCH), jnp.int32),
        pltpu.VMEM((NB, BATCH), jnp.int32),
        pltpu.VMEM((NBUF, BATCH, HALF), jnp.float32),
        pltpu.VMEM_SHARED((ACC_ROWS, HALF), jnp.float32),
        pltpu.SemaphoreType.DMA((2 * NBUF,)),
        pltpu.SemaphoreType.DMA((NBUF,)),
    ],
    compiler_params=pltpu.CompilerParams(needs_layout_passes=False),
)
def _sc_edge(h0_hbm, h1_hbm, src_hbm, dst_hbm, o0_hbm, o1_hbm,
             src_v, dst_v, buf_v, acc_sh, gsem, ssem):
    c = lax.axis_index("c")
    s = lax.axis_index("s")

    def run(h_hbm, o_hbm):
        @pl.when(s < 15)
        def _():
            pltpu.sync_copy(h_hbm.at[pl.ds(s * ROWS_PER_TILE, ROWS_PER_TILE)],
                            acc_sh.at[pl.ds(s * ROWS_PER_TILE, ROWS_PER_TILE)])

        @pl.when(s == 15)
        def _():
            pltpu.sync_copy(h_hbm.at[pl.ds(15 * ROWS_PER_TILE, LAST_ROWS)],
                            acc_sh.at[pl.ds(15 * ROWS_PER_TILE, LAST_ROWS)])
        plsc.subcore_barrier()

        def g_desc(k, slot, half):
            return pltpu.make_async_copy(
                h_hbm.at[src_v.at[2 * k + half]],
                buf_v.at[slot, pl.ds(half * GBATCH, GBATCH)],
                gsem.at[2 * slot + half])

        def s_desc(k, slot):
            return pltpu.make_async_copy(
                buf_v.at[slot], acc_sh.at[dst_v.at[k]], ssem.at[slot])

        def do_chunk(chunk):
            pltpu.sync_copy(src_hbm.at[chunk], src_v)
            pltpu.sync_copy(dst_hbm.at[chunk], dst_v)
            g_desc(0, 0, 0).start()
            g_desc(0, 0, 1).start()

            def body(k, carry):
                slot = lax.bitwise_and(k, NBUF - 1)
                other = 1 - slot
                g_desc(k, slot, 0).wait()
                g_desc(k, slot, 1).wait()
                s_desc(k, slot).start(add=True)

                @pl.when(k >= 1)
                def _():
                    s_desc(k - 1, other).wait()

                @pl.when(k + 1 < NB)
                def _():
                    g_desc(k + 1, other, 0).start()
                    g_desc(k + 1, other, 1).start()
                return carry
            lax.fori_loop(0, NB, body, 0)
            s_desc(NB - 1, (NB - 1) % NBUF).wait()

        do_chunk(s)
        do_chunk(s + 16)
        plsc.subcore_barrier()

        @pl.when(s < 15)
        def _():
            pltpu.sync_copy(acc_sh.at[pl.ds(s * ROWS_PER_TILE, ROWS_PER_TILE)],
                            o_hbm.at[pl.ds(s * ROWS_PER_TILE, ROWS_PER_TILE)])

        @pl.when(s == 15)
        def _():
            pltpu.sync_copy(acc_sh.at[pl.ds(15 * ROWS_PER_TILE, LAST_ROWS)],
                            o_hbm.at[pl.ds(15 * ROWS_PER_TILE, LAST_ROWS)])

    @pl.when(c == 0)
    def _():
        run(h0_hbm, o0_hbm)

    @pl.when(c == 1)
    def _():
        run(h1_hbm, o1_hbm)


def _tc1_body(x_ref, w_ref, d_ref, h0_ref, h1_ref, dv_ref):
    deg = jnp.sum(d_ref[:], axis=1, keepdims=True) + 1.0
    dv = lax.rsqrt(deg)
    h = jnp.dot(x_ref[:], w_ref[:], preferred_element_type=jnp.float32)
    h = h * dv
    h0_ref[:] = h[:, :HALF]
    h1_ref[:] = h[:, HALF:]
    dv_ref[:] = dv


def _tc2_body(a0_ref, a1_ref, dv_ref, b1_ref, w2_ref, g0_ref, g1_ref):
    dv = dv_ref[:]
    xb = jnp.concatenate([a0_ref[:], a1_ref[:]], axis=1) * dv + b1_ref[:][None, :]
    xb = jnp.maximum(xb, 0.0)
    g = jnp.dot(xb, w2_ref[:], preferred_element_type=jnp.float32) * dv
    g0_ref[:] = g[:, :HALF]
    g1_ref[:] = g[:, HALF:]


def _tc3_body(a0_ref, a1_ref, dv_ref, b2_ref, o_ref):
    o_ref[:] = (jnp.concatenate([a0_ref[:], a1_ref[:]], axis=1) * dv_ref[:]
                + b2_ref[:][None, :])


_GRID = N // ROW_BLK

_row_spec = pl.BlockSpec((ROW_BLK, F), lambda i: (i, 0))
_half_spec = pl.BlockSpec((ROW_BLK, HALF), lambda i: (i, 0))
_vec_spec = pl.BlockSpec((ROW_BLK, 1), lambda i: (i, 0))
_w_spec = pl.BlockSpec((F, F), lambda i: (0, 0))
_b_spec = pl.BlockSpec((F,), lambda i: (0,))

_deg_spec = pl.BlockSpec((ROW_BLK, NCHUNK), lambda i: (i, 0))

_tc1 = pl.pallas_call(
    _tc1_body,
    grid=(_GRID,),
    in_specs=[_row_spec, _w_spec, _deg_spec],
    out_specs=(_half_spec, _half_spec, _vec_spec),
    out_shape=(
        jax.ShapeDtypeStruct((N, HALF), jnp.float32),
        jax.ShapeDtypeStruct((N, HALF), jnp.float32),
        jax.ShapeDtypeStruct((N, 1), jnp.float32),
    ),
)

_tc2 = pl.pallas_call(
    _tc2_body,
    grid=(_GRID,),
    in_specs=[_half_spec, _half_spec, _vec_spec, _b_spec, _w_spec],
    out_specs=(_half_spec, _half_spec),
    out_shape=(
        jax.ShapeDtypeStruct((N, HALF), jnp.float32),
        jax.ShapeDtypeStruct((N, HALF), jnp.float32),
    ),
)

_tc3 = pl.pallas_call(
    _tc3_body,
    grid=(_GRID,),
    in_specs=[_half_spec, _half_spec, _vec_spec, _b_spec],
    out_specs=_row_spec,
    out_shape=jax.ShapeDtypeStruct((N, F), jnp.float32),
)


def kernel(x, W1, b1, W2, b2, edge_index):
    src = edge_index[0].astype(jnp.int32)
    dst = edge_index[1].astype(jnp.int32)
    srcp = jnp.pad(src.reshape(NCHUNK, CHUNK),
                   ((0, 0), (0, CHUNK_PAD - CHUNK))).reshape(NCHUNK, NB, BATCH)
    dstp = jnp.pad(dst.reshape(NCHUNK, CHUNK),
                   ((0, 0), (0, CHUNK_PAD - CHUNK)),
                   constant_values=N).reshape(NCHUNK, NB, BATCH)

    srcg = srcp.reshape(NCHUNK, GNB, GBATCH)

    dparts = _sc_deg(dstp)
    dparts = dparts.T[:N]
    h0, h1, dv = _tc1(x, W1, dparts)
    a0, a1 = _sc_edge(h0, h1, srcg, dstp)
    g0, g1 = _tc2(a0, a1, dv, b1, W2)
    o0, o1 = _sc_edge(g0, g1, srcg, dstp)
    return _tc3(o0, o1, dv, b2)


# speedup vs baseline: 9.6912x; 1.0004x over previous
import functools

import jax
import jax.numpy as jnp
from jax import lax
from jax.experimental import pallas as pl
from jax.experimental.pallas import tpu as pltpu
from jax.experimental.pallas import tpu_sc as plsc

N = 10000
F = 256
HALF = 128
E = 160000
NCHUNK = 32
CHUNK = 5000
CHUNK_PAD = 5120
NB = 40
BATCH = 128
ACC_ROWS = 10016
ROWS_PER_TILE = 632
LAST_ROWS = N - 15 * ROWS_PER_TILE
GBATCH = 128
GNB = CHUNK_PAD // GBATCH
NBUF = 2
DEG_ROWS = 80
ROW_BLK = 1000

_mesh = plsc.VectorSubcoreMesh(core_axis_name="c", subcore_axis_name="s")


HIST = DEG_ROWS * BATCH


@functools.partial(
    pl.kernel,
    mesh=_mesh,
    out_type=jax.ShapeDtypeStruct((NCHUNK, HIST), jnp.float32),
    scratch_types=[
        pltpu.VMEM((NB, BATCH), jnp.int32),
        pltpu.VMEM((HIST,), jnp.float32),
    ],
    compiler_params=pltpu.CompilerParams(needs_layout_passes=False),
)
def _sc_deg(dst_hbm, d_hbm, dst_v, hist_v):
    c = lax.axis_index("c")
    s = lax.axis_index("s")
    chunk = c * 16 + s
    pltpu.sync_copy(dst_hbm.at[chunk], dst_v)

    zeros16 = jnp.zeros((16,), jnp.float32)
    ones16 = jnp.ones((16,), jnp.float32)

    def zero_body(i, carry):
        hist_v[pl.ds(i * 16, 16)] = zeros16
        return carry
    lax.fori_loop(0, HIST // 16, zero_body, 0)

    def acc_body(i, carry):
        idx = dst_v[i // 8, pl.ds((i % 8) * 16, 16)]
        plsc.addupdate_scatter(hist_v, [idx], ones16)
        return carry
    lax.fori_loop(0, NB * 8, acc_body, 0)

    pltpu.sync_copy(hist_v, d_hbm.at[chunk])


@functools.partial(
    pl.kernel,
    mesh=_mesh,
    out_type=(
        jax.ShapeDtypeStruct((N, HALF), jnp.float32),
        jax.ShapeDtypeStruct((N, HALF), jnp.float32),
    ),
    scratch_types=[
        pltpu.VMEM((GNB, GBATCH), jnp.int32),
        pltpu.VMEM((GNB, GBATCH), jnp.int32),
        pltpu.VMEM((NBUF, GBATCH, HALF), jnp.float32),
        pltpu.VMEM_SHARED((ACC_ROWS, HALF), jnp.float32),
        pltpu.SemaphoreType.DMA((NBUF,)),
        pltpu.SemaphoreType.DMA((NBUF,)),
    ],
    compiler_params=pltpu.CompilerParams(needs_layout_passes=False),
)
def _sc_edge(h0_hbm, h1_hbm, src_hbm, dst_hbm, o0_hbm, o1_hbm,
             src_v, dst_v, buf_v, acc_sh, gsem, ssem):
    c = lax.axis_index("c")
    s = lax.axis_index("s")

    def run(h_hbm, o_hbm):
        @pl.when(s < 15)
        def _():
            pltpu.sync_copy(h_hbm.at[pl.ds(s * ROWS_PER_TILE, ROWS_PER_TILE)],
                            acc_sh.at[pl.ds(s * ROWS_PER_TILE, ROWS_PER_TILE)])

        @pl.when(s == 15)
        def _():
            pltpu.sync_copy(h_hbm.at[pl.ds(15 * ROWS_PER_TILE, LAST_ROWS)],
                            acc_sh.at[pl.ds(15 * ROWS_PER_TILE, LAST_ROWS)])
        plsc.subcore_barrier()

        def g_desc(j, slot):
            return pltpu.make_async_copy(
                h_hbm.at[src_v.at[j]], buf_v.at[slot], gsem.at[slot])

        def s_desc(j, slot):
            return pltpu.make_async_copy(
                buf_v.at[slot], acc_sh.at[dst_v.at[j]], ssem.at[slot])

        def do_chunk(chunk):
            pltpu.sync_copy(src_hbm.at[chunk], src_v)
            pltpu.sync_copy(dst_hbm.at[chunk], dst_v)
            g_desc(0, 0).start()

            def body(j, carry):
                slot = lax.bitwise_and(j, NBUF - 1)
                other = 1 - slot
                g_desc(j, slot).wait()
                s_desc(j, slot).start(add=True)

                @pl.when(j >= 1)
                def _():
                    s_desc(j - 1, other).wait()

                @pl.when(j + 1 < GNB)
                def _():
                    g_desc(j + 1, other).start()
                return carry
            lax.fori_loop(0, GNB, body, 0)
            s_desc(GNB - 1, (GNB - 1) % NBUF).wait()

        do_chunk(s)
        do_chunk(s + 16)
        plsc.subcore_barrier()

        @pl.when(s < 15)
        def _():
            pltpu.sync_copy(acc_sh.at[pl.ds(s * ROWS_PER_TILE, ROWS_PER_TILE)],
                            o_hbm.at[pl.ds(s * ROWS_PER_TILE, ROWS_PER_TILE)])

        @pl.when(s == 15)
        def _():
            pltpu.sync_copy(acc_sh.at[pl.ds(15 * ROWS_PER_TILE, LAST_ROWS)],
                            o_hbm.at[pl.ds(15 * ROWS_PER_TILE, LAST_ROWS)])

    @pl.when(c == 0)
    def _():
        run(h0_hbm, o0_hbm)

    @pl.when(c == 1)
    def _():
        run(h1_hbm, o1_hbm)


def _tc1_body(x_ref, w_ref, d_ref, h0_ref, h1_ref, dv_ref):
    deg = jnp.sum(d_ref[:], axis=1, keepdims=True) + 1.0
    dv = lax.rsqrt(deg)
    h = jnp.dot(x_ref[:].astype(jnp.bfloat16), w_ref[:].astype(jnp.bfloat16),
                preferred_element_type=jnp.float32)
    h = h * dv
    h0_ref[:] = h[:, :HALF]
    h1_ref[:] = h[:, HALF:]
    dv_ref[:] = dv


def _tc2_body(a0_ref, a1_ref, dv_ref, b1_ref, w2_ref, g0_ref, g1_ref):
    dv = dv_ref[:]
    xb = jnp.concatenate([a0_ref[:], a1_ref[:]], axis=1) * dv + b1_ref[:][None, :]
    xb = jnp.maximum(xb, 0.0)
    g = jnp.dot(xb.astype(jnp.bfloat16), w2_ref[:].astype(jnp.bfloat16),
                preferred_element_type=jnp.float32) * dv
    g0_ref[:] = g[:, :HALF]
    g1_ref[:] = g[:, HALF:]


def _tc3_body(a0_ref, a1_ref, dv_ref, b2_ref, o_ref):
    o_ref[:] = (jnp.concatenate([a0_ref[:], a1_ref[:]], axis=1) * dv_ref[:]
                + b2_ref[:][None, :])


_GRID = N // ROW_BLK

_row_spec = pl.BlockSpec((ROW_BLK, F), lambda i: (i, 0))
_half_spec = pl.BlockSpec((ROW_BLK, HALF), lambda i: (i, 0))
_vec_spec = pl.BlockSpec((ROW_BLK, 1), lambda i: (i, 0))
_w_spec = pl.BlockSpec((F, F), lambda i: (0, 0))
_b_spec = pl.BlockSpec((F,), lambda i: (0,))

_deg_spec = pl.BlockSpec((ROW_BLK, NCHUNK), lambda i: (i, 0))

_tc1 = pl.pallas_call(
    _tc1_body,
    grid=(_GRID,),
    in_specs=[_row_spec, _w_spec, _deg_spec],
    out_specs=(_half_spec, _half_spec, _vec_spec),
    out_shape=(
        jax.ShapeDtypeStruct((N, HALF), jnp.float32),
        jax.ShapeDtypeStruct((N, HALF), jnp.float32),
        jax.ShapeDtypeStruct((N, 1), jnp.float32),
    ),
)

_tc2 = pl.pallas_call(
    _tc2_body,
    grid=(_GRID,),
    in_specs=[_half_spec, _half_spec, _vec_spec, _b_spec, _w_spec],
    out_specs=(_half_spec, _half_spec),
    out_shape=(
        jax.ShapeDtypeStruct((N, HALF), jnp.float32),
        jax.ShapeDtypeStruct((N, HALF), jnp.float32),
    ),
)

_tc3 = pl.pallas_call(
    _tc3_body,
    grid=(_GRID,),
    in_specs=[_half_spec, _half_spec, _vec_spec, _b_spec],
    out_specs=_row_spec,
    out_shape=jax.ShapeDtypeStruct((N, F), jnp.float32),
)


def kernel(x, W1, b1, W2, b2, edge_index):
    src = edge_index[0].astype(jnp.int32)
    dst = edge_index[1].astype(jnp.int32)
    srcp = jnp.pad(src.reshape(NCHUNK, CHUNK),
                   ((0, 0), (0, CHUNK_PAD - CHUNK))).reshape(NCHUNK, NB, BATCH)
    dstp = jnp.pad(dst.reshape(NCHUNK, CHUNK),
                   ((0, 0), (0, CHUNK_PAD - CHUNK)),
                   constant_values=N).reshape(NCHUNK, NB, BATCH)

    srcg = srcp.reshape(NCHUNK, GNB, GBATCH)
    dstg = dstp.reshape(NCHUNK, GNB, GBATCH)

    dparts = _sc_deg(dstp)
    dparts = dparts.T[:N]
    h0, h1, dv = _tc1(x, W1, dparts)
    a0, a1 = _sc_edge(h0, h1, srcg, dstg)
    g0, g1 = _tc2(a0, a1, dv, b1, W2)
    o0, o1 = _sc_edge(g0, g1, srcg, dstg)
    return _tc3(o0, o1, dv, b2)


# speedup vs baseline: 10.5918x; 1.0929x over previous
import functools

import jax
import jax.numpy as jnp
from jax import lax
from jax.experimental import pallas as pl
from jax.experimental.pallas import tpu as pltpu
from jax.experimental.pallas import tpu_sc as plsc

N = 10000
F = 256
HALF = 128
E = 160000
NCHUNK = 32
CHUNK = 5000
CHUNK_PAD = 5120
NB = 40
BATCH = 128
ACC_ROWS = 10016
ROWS_PER_TILE = 632
LAST_ROWS = N - 15 * ROWS_PER_TILE
GBATCH = 80
GNB = CHUNK_PAD // GBATCH
NBUF = 3
DEG_ROWS = 80
ROW_BLK = 1000

_mesh = plsc.VectorSubcoreMesh(core_axis_name="c", subcore_axis_name="s")


HIST = DEG_ROWS * BATCH


@functools.partial(
    pl.kernel,
    mesh=_mesh,
    out_type=jax.ShapeDtypeStruct((NCHUNK, HIST), jnp.float32),
    scratch_types=[
        pltpu.VMEM((NB, BATCH), jnp.int32),
        pltpu.VMEM((HIST,), jnp.float32),
    ],
    compiler_params=pltpu.CompilerParams(needs_layout_passes=False),
)
def _sc_deg(dst_hbm, d_hbm, dst_v, hist_v):
    c = lax.axis_index("c")
    s = lax.axis_index("s")
    chunk = c * 16 + s
    pltpu.sync_copy(dst_hbm.at[chunk], dst_v)

    zeros16 = jnp.zeros((16,), jnp.float32)
    ones16 = jnp.ones((16,), jnp.float32)

    def zero_body(i, carry):
        hist_v[pl.ds(i * 16, 16)] = zeros16
        return carry
    lax.fori_loop(0, HIST // 16, zero_body, 0)

    def acc_body(i, carry):
        idx = dst_v[i // 8, pl.ds((i % 8) * 16, 16)]
        plsc.addupdate_scatter(hist_v, [idx], ones16)
        return carry
    lax.fori_loop(0, NB * 8, acc_body, 0)

    pltpu.sync_copy(hist_v, d_hbm.at[chunk])


@functools.partial(
    pl.kernel,
    mesh=_mesh,
    out_type=(
        jax.ShapeDtypeStruct((N, HALF), jnp.float32),
        jax.ShapeDtypeStruct((N, HALF), jnp.float32),
    ),
    scratch_types=[
        pltpu.VMEM((GNB, GBATCH), jnp.int32),
        pltpu.VMEM((GNB, GBATCH), jnp.int32),
        pltpu.VMEM((NBUF, GBATCH, HALF), jnp.float32),
        pltpu.VMEM_SHARED((ACC_ROWS, HALF), jnp.float32),
        pltpu.SemaphoreType.DMA((NBUF,)),
        pltpu.SemaphoreType.DMA((NBUF,)),

    ],
    compiler_params=pltpu.CompilerParams(needs_layout_passes=False),
)
def _sc_edge(h0_hbm, h1_hbm, src_hbm, dst_hbm, o0_hbm, o1_hbm,
             src_v, dst_v, buf_v, acc_sh, gsem, ssem):
    c = lax.axis_index("c")
    s = lax.axis_index("s")

    def run(h_hbm, o_hbm):
        @pl.when(s < 15)
        def _():
            pltpu.sync_copy(h_hbm.at[pl.ds(s * ROWS_PER_TILE, ROWS_PER_TILE)],
                            acc_sh.at[pl.ds(s * ROWS_PER_TILE, ROWS_PER_TILE)])

        @pl.when(s == 15)
        def _():
            pltpu.sync_copy(h_hbm.at[pl.ds(15 * ROWS_PER_TILE, LAST_ROWS)],
                            acc_sh.at[pl.ds(15 * ROWS_PER_TILE, LAST_ROWS)])
        plsc.subcore_barrier()

        def g_desc(j, slot):
            return pltpu.make_async_copy(
                h_hbm.at[src_v.at[j]], buf_v.at[slot], gsem.at[slot])

        def s_desc(j, slot):
            return pltpu.make_async_copy(
                buf_v.at[slot], acc_sh.at[dst_v.at[j]], ssem.at[slot])

        def do_chunk(chunk):
            pltpu.sync_copy(src_hbm.at[chunk], src_v)
            pltpu.sync_copy(dst_hbm.at[chunk], dst_v)
            g_desc(0, 0).start()
            g_desc(1, 1).start()

            def body(j, carry):
                slot = lax.rem(j, NBUF)
                g_desc(j, slot).wait()
                s_desc(j, slot).start(add=True)

                @pl.when(j >= 1)
                def _():
                    s_desc(j - 1, lax.rem(j - 1, NBUF)).wait()

                @pl.when(j + 2 < GNB)
                def _():
                    g_desc(j + 2, lax.rem(j + 2, NBUF)).start()
                return carry
            lax.fori_loop(0, GNB, body, 0)
            s_desc(GNB - 1, (GNB - 1) % NBUF).wait()

        do_chunk(s)
        do_chunk(s + 16)
        plsc.subcore_barrier()

        @pl.when(s < 15)
        def _():
            pltpu.sync_copy(acc_sh.at[pl.ds(s * ROWS_PER_TILE, ROWS_PER_TILE)],
                            o_hbm.at[pl.ds(s * ROWS_PER_TILE, ROWS_PER_TILE)])

        @pl.when(s == 15)
        def _():
            pltpu.sync_copy(acc_sh.at[pl.ds(15 * ROWS_PER_TILE, LAST_ROWS)],
                            o_hbm.at[pl.ds(15 * ROWS_PER_TILE, LAST_ROWS)])

    @pl.when(c == 0)
    def _():
        run(h0_hbm, o0_hbm)

    @pl.when(c == 1)
    def _():
        run(h1_hbm, o1_hbm)


def _tc1_body(x_ref, w_ref, d_ref, h0_ref, h1_ref, dv_ref):
    deg = jnp.sum(d_ref[:], axis=1, keepdims=True) + 1.0
    dv = lax.rsqrt(deg)
    h = jnp.dot(x_ref[:].astype(jnp.bfloat16), w_ref[:].astype(jnp.bfloat16),
                preferred_element_type=jnp.float32)
    h = h * dv
    h0_ref[:] = h[:, :HALF]
    h1_ref[:] = h[:, HALF:]
    dv_ref[:] = dv


def _tc2_body(a0_ref, a1_ref, dv_ref, b1_ref, w2_ref, g0_ref, g1_ref):
    dv = dv_ref[:]
    xb = jnp.concatenate([a0_ref[:], a1_ref[:]], axis=1) * dv + b1_ref[:][None, :]
    xb = jnp.maximum(xb, 0.0)
    g = jnp.dot(xb.astype(jnp.bfloat16), w2_ref[:].astype(jnp.bfloat16),
                preferred_element_type=jnp.float32) * dv
    g0_ref[:] = g[:, :HALF]
    g1_ref[:] = g[:, HALF:]


def _tc3_body(a0_ref, a1_ref, dv_ref, b2_ref, o_ref):
    o_ref[:] = (jnp.concatenate([a0_ref[:], a1_ref[:]], axis=1) * dv_ref[:]
                + b2_ref[:][None, :])


_GRID = N // ROW_BLK

_row_spec = pl.BlockSpec((ROW_BLK, F), lambda i: (i, 0))
_half_spec = pl.BlockSpec((ROW_BLK, HALF), lambda i: (i, 0))
_vec_spec = pl.BlockSpec((ROW_BLK, 1), lambda i: (i, 0))
_w_spec = pl.BlockSpec((F, F), lambda i: (0, 0))
_b_spec = pl.BlockSpec((F,), lambda i: (0,))

_deg_spec = pl.BlockSpec((ROW_BLK, NCHUNK), lambda i: (i, 0))

_tc1 = pl.pallas_call(
    _tc1_body,
    grid=(_GRID,),
    in_specs=[_row_spec, _w_spec, _deg_spec],
    out_specs=(_half_spec, _half_spec, _vec_spec),
    out_shape=(
        jax.ShapeDtypeStruct((N, HALF), jnp.float32),
        jax.ShapeDtypeStruct((N, HALF), jnp.float32),
        jax.ShapeDtypeStruct((N, 1), jnp.float32),
    ),
)

_tc2 = pl.pallas_call(
    _tc2_body,
    grid=(_GRID,),
    in_specs=[_half_spec, _half_spec, _vec_spec, _b_spec, _w_spec],
    out_specs=(_half_spec, _half_spec),
    out_shape=(
        jax.ShapeDtypeStruct((N, HALF), jnp.float32),
        jax.ShapeDtypeStruct((N, HALF), jnp.float32),
    ),
)

_tc3 = pl.pallas_call(
    _tc3_body,
    grid=(_GRID,),
    in_specs=[_half_spec, _half_spec, _vec_spec, _b_spec],
    out_specs=_row_spec,
    out_shape=jax.ShapeDtypeStruct((N, F), jnp.float32),
)


def kernel(x, W1, b1, W2, b2, edge_index):
    src = edge_index[0].astype(jnp.int32)
    dst = edge_index[1].astype(jnp.int32)
    srcp = jnp.pad(src.reshape(NCHUNK, CHUNK),
                   ((0, 0), (0, CHUNK_PAD - CHUNK))).reshape(NCHUNK, NB, BATCH)
    dstp = jnp.pad(dst.reshape(NCHUNK, CHUNK),
                   ((0, 0), (0, CHUNK_PAD - CHUNK)),
                   constant_values=N).reshape(NCHUNK, NB, BATCH)

    srcg = srcp.reshape(NCHUNK, GNB, GBATCH)
    dstg = dstp.reshape(NCHUNK, GNB, GBATCH)

    dparts = _sc_deg(dstp)
    dparts = dparts.T[:N]
    h0, h1, dv = _tc1(x, W1, dparts)
    a0, a1 = _sc_edge(h0, h1, srcg, dstg)
    g0, g1 = _tc2(a0, a1, dv, b1, W2)
    o0, o1 = _sc_edge(g0, g1, srcg, dstg)
    return _tc3(o0, o1, dv, b2)


# speedup vs baseline: 10.6854x; 1.0088x over previous
import functools

import jax
import jax.numpy as jnp
from jax import lax
from jax.experimental import pallas as pl
from jax.experimental.pallas import tpu as pltpu
from jax.experimental.pallas import tpu_sc as plsc

N = 10000
F = 256
HALF = 128
E = 160000
NCHUNK = 32
CHUNK = 5000
CHUNK_PAD = 5120
NB = 40
BATCH = 128
ACC_ROWS = 10016
ROWS_PER_TILE = 632
LAST_ROWS = N - 15 * ROWS_PER_TILE
GBATCH = 128
GNB = CHUNK_PAD // GBATCH
NBUF = 3
DEG_ROWS = 80
ROW_BLK = 1000

_mesh = plsc.VectorSubcoreMesh(core_axis_name="c", subcore_axis_name="s")


HIST = DEG_ROWS * BATCH


@functools.partial(
    pl.kernel,
    mesh=_mesh,
    out_type=jax.ShapeDtypeStruct((NCHUNK, HIST), jnp.float32),
    scratch_types=[
        pltpu.VMEM((NB, BATCH), jnp.int32),
        pltpu.VMEM((HIST,), jnp.float32),
    ],
    compiler_params=pltpu.CompilerParams(needs_layout_passes=False),
)
def _sc_deg(dst_hbm, d_hbm, dst_v, hist_v):
    c = lax.axis_index("c")
    s = lax.axis_index("s")
    chunk = c * 16 + s
    pltpu.sync_copy(dst_hbm.at[chunk], dst_v)

    zeros16 = jnp.zeros((16,), jnp.float32)
    ones16 = jnp.ones((16,), jnp.float32)

    def zero_body(i, carry):
        hist_v[pl.ds(i * 16, 16)] = zeros16
        return carry
    lax.fori_loop(0, HIST // 16, zero_body, 0)

    def acc_body(i, carry):
        idx = dst_v[i // 8, pl.ds((i % 8) * 16, 16)]
        plsc.addupdate_scatter(hist_v, [idx], ones16)
        return carry
    lax.fori_loop(0, NB * 8, acc_body, 0)

    pltpu.sync_copy(hist_v, d_hbm.at[chunk])


@functools.partial(
    pl.kernel,
    mesh=_mesh,
    out_type=(
        jax.ShapeDtypeStruct((N, HALF), jnp.float32),
        jax.ShapeDtypeStruct((N, HALF), jnp.float32),
    ),
    scratch_types=[
        pltpu.VMEM((4, GBATCH), jnp.int32),
        pltpu.VMEM((4, GBATCH), jnp.int32),
        pltpu.VMEM((NBUF, GBATCH, HALF), jnp.float32),
        pltpu.VMEM_SHARED((ACC_ROWS, HALF), jnp.float32),
        pltpu.SemaphoreType.DMA((NBUF,)),
        pltpu.SemaphoreType.DMA((NBUF,)),
        pltpu.SemaphoreType.DMA((4,)),
        pltpu.SemaphoreType.DMA((4,)),
    ],
    compiler_params=pltpu.CompilerParams(needs_layout_passes=False),
)
def _sc_edge(h0_hbm, h1_hbm, src_hbm, dst_hbm, o0_hbm, o1_hbm,
             src_v, dst_v, buf_v, acc_sh, gsem, ssem, xssem, xdsem):
    c = lax.axis_index("c")
    s = lax.axis_index("s")

    def run(h_hbm, o_hbm):
        @pl.when(s < 15)
        def _():
            pltpu.sync_copy(h_hbm.at[pl.ds(s * ROWS_PER_TILE, ROWS_PER_TILE)],
                            acc_sh.at[pl.ds(s * ROWS_PER_TILE, ROWS_PER_TILE)])

        @pl.when(s == 15)
        def _():
            pltpu.sync_copy(h_hbm.at[pl.ds(15 * ROWS_PER_TILE, LAST_ROWS)],
                            acc_sh.at[pl.ds(15 * ROWS_PER_TILE, LAST_ROWS)])
        plsc.subcore_barrier()

        def g_desc(j, slot):
            return pltpu.make_async_copy(
                h_hbm.at[src_v.at[lax.bitwise_and(j, 3)]],
                buf_v.at[slot], gsem.at[slot])

        def s_desc(j, slot):
            return pltpu.make_async_copy(
                buf_v.at[slot], acc_sh.at[dst_v.at[lax.bitwise_and(j, 3)]],
                ssem.at[slot])

        def do_chunk(chunk):
            def xs_desc(j):
                r = lax.bitwise_and(j, 3)
                return pltpu.make_async_copy(
                    src_hbm.at[chunk, j], src_v.at[r], xssem.at[r])

            def xd_desc(j):
                r = lax.bitwise_and(j, 3)
                return pltpu.make_async_copy(
                    dst_hbm.at[chunk, j], dst_v.at[r], xdsem.at[r])

            for r in range(4):
                xs_desc(r).start()
            for r in range(3):
                xd_desc(r).start()
            xs_desc(0).wait()
            g_desc(0, 0).start()
            xs_desc(1).wait()
            g_desc(1, 1).start()

            def body(j, carry):
                slot = lax.rem(j, NBUF)
                g_desc(j, slot).wait()

                @pl.when(j + 4 < GNB)
                def _():
                    xs_desc(j + 4).start()

                xd_desc(j).wait()
                s_desc(j, slot).start(add=True)

                @pl.when(j >= 1)
                def _():
                    s_desc(j - 1, lax.rem(j - 1, NBUF)).wait()

                @pl.when(j + 3 < GNB)
                def _():
                    xd_desc(j + 3).start()

                @pl.when(j + 2 < GNB)
                def _():
                    xs_desc(j + 2).wait()
                    g_desc(j + 2, lax.rem(j + 2, NBUF)).start()
                return carry
            lax.fori_loop(0, GNB, body, 0)
            s_desc(GNB - 1, (GNB - 1) % NBUF).wait()

        do_chunk(s)
        do_chunk(s + 16)
        plsc.subcore_barrier()

        @pl.when(s < 15)
        def _():
            pltpu.sync_copy(acc_sh.at[pl.ds(s * ROWS_PER_TILE, ROWS_PER_TILE)],
                            o_hbm.at[pl.ds(s * ROWS_PER_TILE, ROWS_PER_TILE)])

        @pl.when(s == 15)
        def _():
            pltpu.sync_copy(acc_sh.at[pl.ds(15 * ROWS_PER_TILE, LAST_ROWS)],
                            o_hbm.at[pl.ds(15 * ROWS_PER_TILE, LAST_ROWS)])

    @pl.when(c == 0)
    def _():
        run(h0_hbm, o0_hbm)

    @pl.when(c == 1)
    def _():
        run(h1_hbm, o1_hbm)


def _tc1_body(x_ref, w_ref, d_ref, h0_ref, h1_ref, dv_ref):
    deg = jnp.sum(d_ref[:], axis=1, keepdims=True) + 1.0
    dv = lax.rsqrt(deg)
    h = jnp.dot(x_ref[:].astype(jnp.bfloat16), w_ref[:].astype(jnp.bfloat16),
                preferred_element_type=jnp.float32)
    h = h * dv
    h0_ref[:] = h[:, :HALF]
    h1_ref[:] = h[:, HALF:]
    dv_ref[:] = dv


def _tc2_body(a0_ref, a1_ref, dv_ref, b1_ref, w2_ref, g0_ref, g1_ref):
    dv = dv_ref[:]
    xb = jnp.concatenate([a0_ref[:], a1_ref[:]], axis=1) * dv + b1_ref[:][None, :]
    xb = jnp.maximum(xb, 0.0)
    g = jnp.dot(xb.astype(jnp.bfloat16), w2_ref[:].astype(jnp.bfloat16),
                preferred_element_type=jnp.float32) * dv
    g0_ref[:] = g[:, :HALF]
    g1_ref[:] = g[:, HALF:]


def _tc3_body(a0_ref, a1_ref, dv_ref, b2_ref, o_ref):
    o_ref[:] = (jnp.concatenate([a0_ref[:], a1_ref[:]], axis=1) * dv_ref[:]
                + b2_ref[:][None, :])


_GRID = N // ROW_BLK

_row_spec = pl.BlockSpec((ROW_BLK, F), lambda i: (i, 0))
_half_spec = pl.BlockSpec((ROW_BLK, HALF), lambda i: (i, 0))
_vec_spec = pl.BlockSpec((ROW_BLK, 1), lambda i: (i, 0))
_w_spec = pl.BlockSpec((F, F), lambda i: (0, 0))
_b_spec = pl.BlockSpec((F,), lambda i: (0,))

_deg_spec = pl.BlockSpec((ROW_BLK, NCHUNK), lambda i: (i, 0))

_tc1 = pl.pallas_call(
    _tc1_body,
    grid=(_GRID,),
    in_specs=[_row_spec, _w_spec, _deg_spec],
    out_specs=(_half_spec, _half_spec, _vec_spec),
    out_shape=(
        jax.ShapeDtypeStruct((N, HALF), jnp.float32),
        jax.ShapeDtypeStruct((N, HALF), jnp.float32),
        jax.ShapeDtypeStruct((N, 1), jnp.float32),
    ),
)

_tc2 = pl.pallas_call(
    _tc2_body,
    grid=(_GRID,),
    in_specs=[_half_spec, _half_spec, _vec_spec, _b_spec, _w_spec],
    out_specs=(_half_spec, _half_spec),
    out_shape=(
        jax.ShapeDtypeStruct((N, HALF), jnp.float32),
        jax.ShapeDtypeStruct((N, HALF), jnp.float32),
    ),
)

_tc3 = pl.pallas_call(
    _tc3_body,
    grid=(_GRID,),
    in_specs=[_half_spec, _half_spec, _vec_spec, _b_spec],
    out_specs=_row_spec,
    out_shape=jax.ShapeDtypeStruct((N, F), jnp.float32),
)


def kernel(x, W1, b1, W2, b2, edge_index):
    src = edge_index[0].astype(jnp.int32)
    dst = edge_index[1].astype(jnp.int32)
    srcp = jnp.pad(src.reshape(NCHUNK, CHUNK),
                   ((0, 0), (0, CHUNK_PAD - CHUNK))).reshape(NCHUNK, NB, BATCH)
    dstp = jnp.pad(dst.reshape(NCHUNK, CHUNK),
                   ((0, 0), (0, CHUNK_PAD - CHUNK)),
                   constant_values=N).reshape(NCHUNK, NB, BATCH)

    srcg = srcp.reshape(NCHUNK, GNB, GBATCH)
    dstg = dstp.reshape(NCHUNK, GNB, GBATCH)

    dparts = _sc_deg(dstp)
    dparts = dparts.T[:N]
    h0, h1, dv = _tc1(x, W1, dparts)
    a0, a1 = _sc_edge(h0, h1, srcg, dstg)
    g0, g1 = _tc2(a0, a1, dv, b1, W2)
    o0, o1 = _sc_edge(g0, g1, srcg, dstg)
    return _tc3(o0, o1, dv, b2)
